# Initial kernel scaffold; baseline (speedup 1.0000x reference)
import jax
import jax.numpy as jnp
from jax.experimental import pallas as pl


def kernel(input, input_aug):
    raise NotImplementedError("write your pallas kernel here")


# speedup vs baseline: 1.9664x; 1.9664x over previous
import functools

import jax
import jax.numpy as jnp
from jax.experimental import pallas as pl
from jax.experimental.pallas import tpu as pltpu

_SMOOTH = 0.1


def _body(x_ref, y_ref, out_ref, *, n_rows_total, block_rows, n_cols):
    a = _SMOOTH / (n_cols - 1.0)
    b = (1.0 - _SMOOTH) - a

    x = x_ref[...]
    y = y_ref[...]

    col = jax.lax.broadcasted_iota(jnp.int32, x.shape, 1)
    m = jnp.max(x, axis=1, keepdims=True)
    ti = jnp.min(jnp.where(x == m, col, n_cols), axis=1, keepdims=True)
    aug_t = jnp.sum(jnp.where(col == ti, y, 0.0), axis=1)

    saug = jnp.sum(y, axis=1)
    my = jnp.max(y, axis=1, keepdims=True)
    ly = jnp.log(jnp.sum(jnp.exp(y - my), axis=1))
    mpl = my[:, 0] + ly

    block_tot = (
        -a * jnp.sum(saug)
        + (a * n_cols + b) * jnp.sum(mpl)
        - b * jnp.sum(aug_t)
    )

    @pl.when(pl.program_id(0) == 0)
    def _():
        out_ref[0, 0] = 0.0

    out_ref[0, 0] += block_tot / n_rows_total


def kernel(input, input_aug):
    n_rows, n_cols = input.shape
    block_rows = 256
    grid = n_rows // block_rows

    out = pl.pallas_call(
        functools.partial(
            _body,
            n_rows_total=float(n_rows),
            block_rows=block_rows,
            n_cols=n_cols,
        ),
        grid=(grid,),
        in_specs=[
            pl.BlockSpec((block_rows, n_cols), lambda i: (i, 0)),
            pl.BlockSpec((block_rows, n_cols), lambda i: (i, 0)),
        ],
        out_specs=pl.BlockSpec(memory_space=pltpu.SMEM),
        out_shape=jax.ShapeDtypeStruct((1, 1), jnp.float32),
    )(input, input_aug)
    return out[0, 0]
